# Initial kernel scaffold; baseline (speedup 1.0000x reference)
import functools

import jax
import jax.numpy as jnp
from jax import lax
from jax.experimental import pallas as pl
from jax.experimental.pallas import tpu as pltpu
from jax.experimental.pallas import tpu_sc as plsc

NC = 2
NS = 16
NW = NC * NS
L = 16

N = 65536
NNZ = 4194304
D = 16
NPAD = N + 512
PAIR_CAP = 128

_mesh = plsc.VectorSubcoreMesh(core_axis_name="c", subcore_axis_name="s")


def _wid():
    return lax.axis_index("c") * NS + lax.axis_index("s")


def _splat_i32(k):
    return jnp.zeros((L,), jnp.int32) + k


def _scan_kernel(b_hbm, rows_hbm, cols_hbm, vals_hbm, diag_hbm, stats_hbm,
                 rows_s, cols_s, vals_s, b_s, pr_s, pv_s, z_s, st_s):
    cid = lax.axis_index("c")
    sid = lax.axis_index("s")
    wid = _wid()

    zero = jnp.zeros((L,), jnp.float32)
    sent = _splat_i32(N)

    zc = NPAD // NS

    def zinit(i, _):
        z_s[pl.ds(i * L, L)] = zero
        return 0
    lax.fori_loop(0, zc // L, zinit, 0)
    pltpu.sync_copy(z_s, diag_hbm.at[cid, pl.ds(sid * zc, zc)])

    def pinit(i, _):
        pr_s[pl.ds(i * L, L)] = sent
        pv_s[pl.ds(i * L, L)] = zero
        return 0
    lax.fori_loop(0, PAIR_CAP // L, pinit, 0)
    plsc.subcore_barrier()

    cb = N // NW
    pltpu.sync_copy(b_hbm.at[pl.ds(wid * cb, cb)], b_s)

    def bmax_body(i, acc):
        return jnp.maximum(acc, jnp.abs(b_s[pl.ds(i * L, L)]))
    bmax = lax.fori_loop(0, cb // L, bmax_body, zero)

    ec = NNZ // NW
    kb = rows_s.shape[0]
    nblk = ec // kb

    def flush(cnt):
        pltpu.sync_copy(pv_s, diag_hbm.at[cid].at[pr_s])

        def reinit(i, _):
            pr_s[pl.ds(i * L, L)] = sent
            return 0
        lax.fori_loop(0, PAIR_CAP // L, reinit, 0)
        return cnt * 0

    def blk_body(blk, carry):
        vmax, cnt = carry
        e0 = wid * ec + blk * kb
        pltpu.sync_copy(rows_hbm.at[0, pl.ds(e0, kb)], rows_s)
        pltpu.sync_copy(cols_hbm.at[1, pl.ds(e0, kb)], cols_s)
        pltpu.sync_copy(vals_hbm.at[pl.ds(e0, kb)], vals_s)

        def vec_body(j, carry2):
            vmax2, cnt2 = carry2
            r = rows_s[pl.ds(j * L, L)]
            c = cols_s[pl.ds(j * L, L)]
            v = vals_s[pl.ds(j * L, L)]
            vmax2 = jnp.maximum(vmax2, jnp.abs(v))
            m = r == c

            def hit_branch(cnt3):
                pop = plsc.all_reduce_population_count(m)
                npop = jnp.max(pop)
                plsc.store_compressed(pr_s.at[pl.ds(cnt3, L)], r, mask=m)
                plsc.store_compressed(pv_s.at[pl.ds(cnt3, L)], v, mask=m)
                cnt3 = cnt3 + npop
                return lax.cond(cnt3 > PAIR_CAP - L, flush, lambda x: x, cnt3)

            cnt2 = lax.cond(jnp.any(m), hit_branch, lambda x: x, cnt2)
            return vmax2, cnt2

        return lax.fori_loop(0, kb // L, vec_body, (vmax, cnt))

    vmax, cnt = lax.fori_loop(0, nblk, blk_body,
                              (zero, jnp.zeros((), jnp.int32)))
    flush(cnt)

    st_s[0] = vmax
    st_s[1] = bmax
    pltpu.sync_copy(st_s, stats_hbm.at[wid])


def _stage_a(b, mi, mv):
    kern = pl.kernel(
        _scan_kernel,
        out_type=(
            jax.ShapeDtypeStruct((NC, NPAD), jnp.float32),
            jax.ShapeDtypeStruct((NW, 2, L), jnp.float32),
        ),
        mesh=_mesh,
        scratch_types=[
            pltpu.VMEM((2048,), jnp.int32),
            pltpu.VMEM((2048,), jnp.int32),
            pltpu.VMEM((2048,), jnp.float32),
            pltpu.VMEM((N // NW,), jnp.float32),
            pltpu.VMEM((PAIR_CAP,), jnp.int32),
            pltpu.VMEM((PAIR_CAP,), jnp.float32),
            pltpu.VMEM((NPAD // NS,), jnp.float32),
            pltpu.VMEM((2, L), jnp.float32),
        ],
    )
    return kern(b, mi, mi, mv)


def _h_body(stats_ref, b_ref, diag_ref, w1_ref, h_ref):
    m_max = jnp.maximum(jnp.max(stats_ref[:, 0, :]), 1e-16)
    b_max = jnp.maximum(jnp.max(stats_ref[:, 1, :]), 1e-16)
    d0 = diag_ref[0, :N]
    d1 = diag_ref[1, :N]
    diag = jnp.where(d1 != 0.0, d1, d0)
    xb = b_ref[...] / b_max
    xd = diag / m_max
    w1 = w1_ref[...]
    h = jnp.maximum(xb[:, None] * w1[0][None, :]
                    + xd[:, None] * w1[1][None, :], 0.0)
    h_ref[...] = h


def _stage_h(stats, b, diag, W1):
    return pl.pallas_call(
        _h_body,
        out_shape=jax.ShapeDtypeStruct((N, D), jnp.float32),
    )(stats, b, diag, W1)


def _agg_kernel(rows_hbm, cols_hbm, vals_hbm, h_hbm, agg_hbm,
                rows_s, cols_s, vals_s, hrow_s, msg_s, zrow_s, agg_sh, sem):
    cid = lax.axis_index("c")
    sid = lax.axis_index("s")
    wid = _wid()
    kb = rows_s.shape[0]

    zero = jnp.zeros((L,), jnp.float32)

    def zinit(i, _):
        zrow_s[pl.ds(i * L, L)] = zero
        return 0
    lax.fori_loop(0, (kb * D) // L, zinit, 0)
    rows_per_tile = N // NS

    def zspmem(i, _):
        pltpu.sync_copy(zrow_s.reshape(kb, D),
                        agg_sh.at[pl.ds(sid * rows_per_tile + i * kb, kb)])
        return 0
    lax.fori_loop(0, rows_per_tile // kb, zspmem, 0)
    plsc.subcore_barrier()

    ec = NNZ // NW
    nblk = ec // kb

    def blk_body(blk, _):
        e0 = wid * ec + blk * kb
        pltpu.sync_copy(rows_hbm.at[0, pl.ds(e0, kb)], rows_s)
        pltpu.sync_copy(cols_hbm.at[1, pl.ds(e0, kb)], cols_s)
        pltpu.sync_copy(vals_hbm.at[pl.ds(e0, kb)], vals_s)
        pltpu.async_copy(h_hbm.at[cols_s], hrow_s, sem).wait()

        def edge_body(k, _2):
            val = plsc.load_gather(vals_s, [_splat_i32(k)])
            hv = hrow_s[k]
            msg_s[k] = val * hv
            return 0
        lax.fori_loop(0, kb, edge_body, 0, unroll=8)
        pltpu.sync_copy(msg_s, agg_sh.at[rows_s], add=True)
        return 0

    lax.fori_loop(0, nblk, blk_body, 0)
    plsc.subcore_barrier()

    def wout(i, _):
        pltpu.sync_copy(agg_sh.at[pl.ds(sid * rows_per_tile + i * kb, kb)],
                        agg_hbm.at[cid, pl.ds(sid * rows_per_tile + i * kb, kb)])
        return 0
    lax.fori_loop(0, rows_per_tile // kb, wout, 0)


def _stage_b(mi, mv, h):
    kb = 128
    kern = pl.kernel(
        _agg_kernel,
        out_type=jax.ShapeDtypeStruct((NC, N, D), jnp.float32),
        mesh=_mesh,
        scratch_types=[
            pltpu.VMEM((kb,), jnp.int32),
            pltpu.VMEM((kb,), jnp.int32),
            pltpu.VMEM((kb,), jnp.float32),
            pltpu.VMEM((kb, D), jnp.float32),
            pltpu.VMEM((kb, D), jnp.float32),
            pltpu.VMEM((kb * D,), jnp.float32),
            pltpu.VMEM_SHARED((N, D), jnp.float32),
            pltpu.SemaphoreType.DMA,
        ],
    )
    return kern(mi, mi, mv, h)


def _c_body(stats_ref, h_ref, agg_ref, w2_ref, w3_ref, y_ref):
    m_max = jnp.maximum(jnp.max(stats_ref[:, 0, :]), 1e-16)
    hin = h_ref[...] + (agg_ref[0] + agg_ref[1]) * (1.0 / m_max)
    h2 = jnp.maximum(jnp.dot(hin, w2_ref[...],
                             preferred_element_type=jnp.float32), 0.0)
    y_ref[...] = jnp.dot(h2, w3_ref[...][:, None],
                         preferred_element_type=jnp.float32)[:, 0]


def _stage_c(stats, h, agg, W2, w3):
    return pl.pallas_call(
        _c_body,
        out_shape=jax.ShapeDtypeStruct((N,), jnp.float32),
    )(stats, h, agg, W2, w3)


def _p_kernel(rows_hbm, cols_hbm, vals_hbm, y_hbm, p_hbm,
              rows_s, cols_s, vals_s, yv_s, p_s, y_sh, sem):
    sid = lax.axis_index("s")
    wid = _wid()
    kb = rows_s.shape[0]

    zero = jnp.zeros((L,), jnp.float32)

    def zinit(i, _):
        p_s[pl.ds(i * L, L)] = zero
        return 0
    lax.fori_loop(0, N // L, zinit, 0)

    rows_per_tile = N // NS
    pltpu.sync_copy(y_hbm.at[pl.ds(sid * rows_per_tile, rows_per_tile)],
                    y_sh.at[pl.ds(sid * rows_per_tile, rows_per_tile)])
    plsc.subcore_barrier()

    ec = NNZ // NW
    nblk = ec // kb

    def blk_body(blk, _):
        e0 = wid * ec + blk * kb
        pltpu.sync_copy(rows_hbm.at[0, pl.ds(e0, kb)], rows_s)
        pltpu.sync_copy(cols_hbm.at[1, pl.ds(e0, kb)], cols_s)
        pltpu.sync_copy(vals_hbm.at[pl.ds(e0, kb)], vals_s)
        pltpu.async_copy(y_sh.at[cols_s], yv_s, sem).wait()

        def vec_body(j, _2):
            r = rows_s[pl.ds(j * L, L)]
            prod = vals_s[pl.ds(j * L, L)] * yv_s[pl.ds(j * L, L)]
            plsc.addupdate_scatter(p_s, [r], prod)
            return 0
        lax.fori_loop(0, kb // L, vec_body, 0)
        return 0

    lax.fori_loop(0, nblk, blk_body, 0)
    pltpu.sync_copy(p_s, p_hbm.at[wid])


def _stage_d(mi, mv, y):
    kb = 128
    kern = pl.kernel(
        _p_kernel,
        out_type=jax.ShapeDtypeStruct((NW, N), jnp.float32),
        mesh=_mesh,
        scratch_types=[
            pltpu.VMEM((kb,), jnp.int32),
            pltpu.VMEM((kb,), jnp.int32),
            pltpu.VMEM((kb,), jnp.float32),
            pltpu.VMEM((kb,), jnp.float32),
            pltpu.VMEM((N,), jnp.float32),
            pltpu.VMEM_SHARED((N,), jnp.float32),
            pltpu.SemaphoreType.DMA,
        ],
    )
    return kern(mi, mi, mv, y)


def _e_body(pp_ref, b_ref, y_ref, out_ref):
    p = jnp.sum(pp_ref[...], axis=0)
    psq = jnp.sum(p * p)
    bp = jnp.sum(p * b_ref[...])
    scaler = jnp.maximum(bp / jnp.maximum(psq, 1e-16), 1e-16)
    out_ref[...] = y_ref[...] * scaler


def _stage_e(p_part, b, y):
    return pl.pallas_call(
        _e_body,
        out_shape=jax.ShapeDtypeStruct((N,), jnp.float32),
    )(p_part, b, y)


def kernel(b, m_indices, m_values, W1, W2, w3):
    diag, stats = _stage_a(b, m_indices, m_values)
    h = _stage_h(stats, b, diag, W1)
    agg = _stage_b(m_indices, m_values, h)
    y = _stage_c(stats, h, agg, W2, w3)
    p_part = _stage_d(m_indices, m_values, y)
    return _stage_e(p_part, b, y)


# speedup vs baseline: 12.2248x; 12.2248x over previous
import functools

import jax
import jax.numpy as jnp
from jax import lax
from jax.experimental import pallas as pl
from jax.experimental.pallas import tpu as pltpu
from jax.experimental.pallas import tpu_sc as plsc

NC = 2
NS = 16
NW = NC * NS
L = 16

N = 65536
NNZ = 4194304
D = 16
NPAD = N + 512
PAIR_CAP = 128

_mesh = plsc.VectorSubcoreMesh(core_axis_name="c", subcore_axis_name="s")


def _wid():
    return lax.axis_index("c") * NS + lax.axis_index("s")


def _splat_i32(k):
    return jnp.zeros((L,), jnp.int32) + k


def _scan_kernel(b_hbm, rows_hbm, cols_hbm, vals_hbm, diag_hbm, stats_hbm,
                 rows_s, cols_s, vals_s, b_s, dslab_s, st_s):
    wid = _wid()
    zero = jnp.zeros((L,), jnp.float32)

    def zinit(i, _):
        dslab_s[pl.ds(i * L, L)] = zero
        return 0
    lax.fori_loop(0, N // L, zinit, 0)

    cb = N // NW
    pltpu.sync_copy(b_hbm.at[pl.ds(wid * cb, cb)], b_s)

    def bmax_body(i, acc):
        return jnp.maximum(acc, jnp.abs(b_s[pl.ds(i * L, L)]))
    bmax = lax.fori_loop(0, cb // L, bmax_body, zero)

    ec = NNZ // NW
    kb = rows_s.shape[0]
    nblk = ec // kb

    def blk_body(blk, vmax):
        e0 = wid * ec + blk * kb
        pltpu.sync_copy(rows_hbm.at[0, pl.ds(e0, kb)], rows_s)
        pltpu.sync_copy(cols_hbm.at[1, pl.ds(e0, kb)], cols_s)
        pltpu.sync_copy(vals_hbm.at[pl.ds(e0, kb)], vals_s)

        def vec_body(j, vmax2):
            r = rows_s[pl.ds(j * L, L)]
            c = cols_s[pl.ds(j * L, L)]
            v = vals_s[pl.ds(j * L, L)]
            plsc.store_scatter(dslab_s, [r], v, mask=r == c)
            return jnp.maximum(vmax2, jnp.abs(v))

        return lax.fori_loop(0, kb // L, vec_body, vmax, unroll=4)

    vmax = lax.fori_loop(0, nblk, blk_body, zero)

    pltpu.sync_copy(dslab_s, diag_hbm.at[pl.ds(wid * N, N)])
    st_s[pl.ds(0, L)] = vmax
    st_s[pl.ds(L, L)] = bmax
    pltpu.sync_copy(st_s, stats_hbm.at[pl.ds(wid * 2 * L, 2 * L)])


def _stage_a(b, mi, mv):
    kern = pl.kernel(
        _scan_kernel,
        out_type=(
            jax.ShapeDtypeStruct((NW * N,), jnp.float32),
            jax.ShapeDtypeStruct((NW * 2 * L,), jnp.float32),
        ),
        mesh=_mesh,
        compiler_params=pltpu.CompilerParams(needs_layout_passes=False),
        scratch_types=[
            pltpu.VMEM((2048,), jnp.int32),
            pltpu.VMEM((2048,), jnp.int32),
            pltpu.VMEM((2048,), jnp.float32),
            pltpu.VMEM((N // NW,), jnp.float32),
            pltpu.VMEM((N,), jnp.float32),
            pltpu.VMEM((2 * L,), jnp.float32),
        ],
    )
    return kern(b, mi, mi, mv)


def _lane_max(vec, red_s):
    iota = lax.iota(jnp.int32, L)
    cur = vec
    for stride in (8, 4, 2, 1):
        red_s[pl.ds(0, L)] = cur
        cur = jnp.maximum(cur, plsc.load_gather(red_s, [iota ^ stride]))
    return cur


def _h_kernel(b_hbm, slabs_hbm, stats_hbm, xb_hbm, xd_hbm,
              b_s, sl_s, dm_s, st_s, red_s):
    wid = _wid()
    cb = N // NW

    pltpu.sync_copy(stats_hbm, st_s)
    pltpu.sync_copy(b_hbm.at[pl.ds(wid * cb, cb)], b_s)

    vmax = jnp.zeros((L,), jnp.float32)
    bmax = jnp.zeros((L,), jnp.float32)
    for t in range(NW):
        vmax = jnp.maximum(vmax, st_s[pl.ds(t * 2 * L, L)])
        bmax = jnp.maximum(bmax, st_s[pl.ds(t * 2 * L + L, L)])
    vmax = jnp.maximum(_lane_max(vmax, red_s), 1e-16)
    bmax = jnp.maximum(_lane_max(bmax, red_s), 1e-16)
    inv_m = 1.0 / vmax
    inv_b = 1.0 / bmax

    def mz(i, _):
        dm_s[pl.ds(i * L, L)] = jnp.zeros((L,), jnp.float32)
        return 0
    lax.fori_loop(0, cb // L, mz, 0)
    for t in range(NW):
        pltpu.sync_copy(slabs_hbm.at[pl.ds(t * N + wid * cb, cb)], sl_s)

        def mrg(i, _):
            nv = sl_s[pl.ds(i * L, L)]
            ov = dm_s[pl.ds(i * L, L)]
            dm_s[pl.ds(i * L, L)] = jnp.where(nv != 0.0, nv, ov)
            return 0
        lax.fori_loop(0, cb // L, mrg, 0, unroll=4)

    def scl(i, _):
        b_s[pl.ds(i * L, L)] = b_s[pl.ds(i * L, L)] * inv_b
        dm_s[pl.ds(i * L, L)] = dm_s[pl.ds(i * L, L)] * inv_m
        return 0
    lax.fori_loop(0, cb // L, scl, 0, unroll=4)
    pltpu.sync_copy(b_s, xb_hbm.at[pl.ds(wid * cb, cb)])
    pltpu.sync_copy(dm_s, xd_hbm.at[pl.ds(wid * cb, cb)])


def _stage_h(stats, b, slabs):
    cb = N // NW
    kern = pl.kernel(
        _h_kernel,
        out_type=(
            jax.ShapeDtypeStruct((N,), jnp.float32),
            jax.ShapeDtypeStruct((N,), jnp.float32),
        ),
        mesh=_mesh,
        compiler_params=pltpu.CompilerParams(needs_layout_passes=False),
        scratch_types=[
            pltpu.VMEM((cb,), jnp.float32),
            pltpu.VMEM((cb,), jnp.float32),
            pltpu.VMEM((cb,), jnp.float32),
            pltpu.VMEM((NW * 2 * L,), jnp.float32),
            pltpu.VMEM((L,), jnp.float32),
        ],
    )
    return kern(b, slabs, stats)


def _agg_kernel(rows_hbm, cols_hbm, vals_hbm, xb_hbm, xd_hbm, w1_hbm,
                agg_hbm,
                rows_s, cols_s, vals_s, xbb_s, xdb_s, msg1_s, idx1_s, z1_s,
                w1_s, agg_sh, sem):
    cid = lax.axis_index("c")
    sid = lax.axis_index("s")
    wid = _wid()
    kb = rows_s.shape[0]
    kw = kb * D
    zero = jnp.zeros((L,), jnp.float32)
    iota = lax.iota(jnp.int32, L)

    pltpu.sync_copy(w1_hbm, w1_s)

    def zinit(i, _):
        z1_s[pl.ds(i * L, L)] = zero
        return 0
    lax.fori_loop(0, kw // L, zinit, 0)

    wpt = (N * D) // NS

    def zspmem(i, _):
        pltpu.sync_copy(z1_s, agg_sh.at[pl.ds(sid * wpt + i * kw, kw)])
        return 0
    lax.fori_loop(0, wpt // kw, zspmem, 0)
    plsc.subcore_barrier()

    w10 = w1_s[0]
    w11 = w1_s[1]
    ec = NNZ // NW
    nblk = ec // kb

    def blk_body(blk, _):
        e0 = wid * ec + blk * kb
        pltpu.sync_copy(rows_hbm.at[0, pl.ds(e0, kb)], rows_s)
        pltpu.sync_copy(cols_hbm.at[1, pl.ds(e0, kb)], cols_s)
        pltpu.sync_copy(vals_hbm.at[pl.ds(e0, kb)], vals_s)
        pltpu.async_copy(xb_hbm.at[cols_s], xbb_s, sem).wait()
        pltpu.async_copy(xd_hbm.at[cols_s], xdb_s, sem).wait()

        def edge_body(k, _2):
            ks = _splat_i32(k)
            xbs = plsc.load_gather(xbb_s, [ks])
            xds = plsc.load_gather(xdb_s, [ks])
            val = plsc.load_gather(vals_s, [ks])
            rs = plsc.load_gather(rows_s, [ks])
            hv = jnp.maximum(xbs * w10 + xds * w11, 0.0)
            msg1_s[pl.ds(k * D, D)] = val * hv
            idx1_s[pl.ds(k * D, D)] = rs * D + iota
            return 0
        lax.fori_loop(0, kb, edge_body, 0, unroll=8)
        pltpu.sync_copy(msg1_s, agg_sh.at[idx1_s], add=True)
        return 0

    lax.fori_loop(0, nblk, blk_body, 0)
    plsc.subcore_barrier()

    def wout(i, _):
        off = sid * wpt + i * kw
        pltpu.sync_copy(agg_sh.at[pl.ds(off, kw)],
                        agg_hbm.at[pl.ds(cid * N * D + off, kw)])
        return 0
    lax.fori_loop(0, wpt // kw, wout, 0)


def _stage_b(mi, mv, xb, xd, W1):
    kb = 128
    kern = pl.kernel(
        _agg_kernel,
        out_type=jax.ShapeDtypeStruct((NC * N * D,), jnp.float32),
        mesh=_mesh,
        compiler_params=pltpu.CompilerParams(needs_layout_passes=False),
        scratch_types=[
            pltpu.VMEM((kb,), jnp.int32),
            pltpu.VMEM((kb,), jnp.int32),
            pltpu.VMEM((kb,), jnp.float32),
            pltpu.VMEM((kb,), jnp.float32),
            pltpu.VMEM((kb,), jnp.float32),
            pltpu.VMEM((kb * D,), jnp.float32),
            pltpu.VMEM((kb * D,), jnp.int32),
            pltpu.VMEM((kb * D,), jnp.float32),
            pltpu.VMEM((2, L), jnp.float32),
            pltpu.VMEM_SHARED((N * D,), jnp.float32),
            pltpu.SemaphoreType.DMA,
        ],
    )
    return kern(mi, mi, mv, xb, xd, W1)


_CBLK = 4096


def _c_body(stats_ref, xb_ref, xd_ref, agg_ref, w1_ref, w2_ref, w3_ref,
            y_ref):
    m_max = jnp.maximum(jnp.max(stats_ref[:, 0, :]), 1e-16)
    w1 = w1_ref[...]
    h = jnp.maximum(xb_ref[...][:, None] * w1[0][None, :]
                    + xd_ref[...][:, None] * w1[1][None, :], 0.0)
    a = agg_ref[...]
    hin = h + (a[0] + a[1]) * (1.0 / m_max)
    h2 = jnp.maximum(jnp.dot(hin, w2_ref[...],
                             preferred_element_type=jnp.float32), 0.0)
    y_ref[...] = jnp.dot(h2, w3_ref[...][:, None],
                         preferred_element_type=jnp.float32)[:, 0]


def _stage_c(stats, xb, xd, agg, W1, W2, w3):
    return pl.pallas_call(
        _c_body,
        grid=(N // _CBLK,),
        in_specs=[
            pl.BlockSpec((NW, 2, L), lambda i: (0, 0, 0)),
            pl.BlockSpec((_CBLK,), lambda i: (i,)),
            pl.BlockSpec((_CBLK,), lambda i: (i,)),
            pl.BlockSpec((NC, _CBLK, D), lambda i: (0, i, 0)),
            pl.BlockSpec((2, L), lambda i: (0, 0)),
            pl.BlockSpec((L, L), lambda i: (0, 0)),
            pl.BlockSpec((L,), lambda i: (0,)),
        ],
        out_specs=pl.BlockSpec((_CBLK,), lambda i: (i,)),
        out_shape=jax.ShapeDtypeStruct((N,), jnp.float32),
    )(stats, xb, xd, agg.reshape(NC, N, D), W1, W2, w3)


def _p_kernel(rows_hbm, cols_hbm, vals_hbm, y_hbm, p_hbm,
              rows_s, cols_s, vals_s, yv_s, p_s, sem):
    wid = _wid()
    kb = rows_s.shape[0]

    zero = jnp.zeros((L,), jnp.float32)

    def zinit(i, _):
        p_s[pl.ds(i * L, L)] = zero
        return 0
    lax.fori_loop(0, N // L, zinit, 0)

    ec = NNZ // NW
    nblk = ec // kb

    def blk_body(blk, _):
        e0 = wid * ec + blk * kb
        pltpu.sync_copy(rows_hbm.at[0, pl.ds(e0, kb)], rows_s)
        pltpu.sync_copy(cols_hbm.at[1, pl.ds(e0, kb)], cols_s)
        pltpu.sync_copy(vals_hbm.at[pl.ds(e0, kb)], vals_s)
        pltpu.async_copy(y_hbm.at[cols_s], yv_s, sem).wait()

        def vec_body(j, _2):
            r = rows_s[pl.ds(j * L, L)]
            prod = vals_s[pl.ds(j * L, L)] * yv_s[pl.ds(j * L, L)]
            plsc.addupdate_scatter(p_s, [r], prod)
            return 0
        lax.fori_loop(0, kb // L, vec_body, 0)
        return 0

    lax.fori_loop(0, nblk, blk_body, 0)
    pltpu.sync_copy(p_s, p_hbm.at[pl.ds(wid * N, N)])


def _stage_d(mi, mv, y):
    kb = 128
    kern = pl.kernel(
        _p_kernel,
        out_type=jax.ShapeDtypeStruct((NW * N,), jnp.float32),
        mesh=_mesh,
        compiler_params=pltpu.CompilerParams(needs_layout_passes=False),
        scratch_types=[
            pltpu.VMEM((kb,), jnp.int32),
            pltpu.VMEM((kb,), jnp.int32),
            pltpu.VMEM((kb,), jnp.float32),
            pltpu.VMEM((kb,), jnp.float32),
            pltpu.VMEM((N,), jnp.float32),
            pltpu.SemaphoreType.DMA,
        ],
    )
    return kern(mi, mi, mv, y)


def _e_body(pp_ref, b_ref, y_ref, out_ref):
    p = jnp.sum(pp_ref[...], axis=0)
    psq = jnp.sum(p * p)
    bp = jnp.sum(p * b_ref[...])
    scaler = jnp.maximum(bp / jnp.maximum(psq, 1e-16), 1e-16)
    out_ref[...] = y_ref[...] * scaler


def _stage_e(p_part, b, y):
    return pl.pallas_call(
        _e_body,
        out_shape=jax.ShapeDtypeStruct((N,), jnp.float32),
    )(p_part, b, y)


def kernel(b, m_indices, m_values, W1, W2, w3):
    slabs, stats = _stage_a(b, m_indices, m_values)
    xb, xd = _stage_h(stats, b, slabs)
    agg = _stage_b(m_indices, m_values, xb, xd, W1)
    y = _stage_c(stats.reshape(NW, 2, L), xb, xd, agg, W1, W2, w3)
    p_part = _stage_d(m_indices, m_values, y)
    return _stage_e(p_part.reshape(NW, N), b, y)
